# baseline (speedup 1.0000x reference)
import functools

import jax
import jax.numpy as jnp
from jax import lax
from jax.experimental import pallas as pl
from jax.experimental.pallas import tpu as pltpu
from jax.experimental.pallas import tpu_sc as plsc

HID = 64
W80 = 80
NDEV = 5
NT = 50000
NPAD = 50176
H2 = NPAD // 2
E = 800000
NSUB = 16
PER_SUB = 50176
EPAD = PER_SUB * NSUB
EPAD4 = EPAD * 4
PER_A = EPAD // 32
CHA = 512
CHB = 32
NCHB = PER_SUB // CHB
NZCH = H2 // CHB


def _leaky(x, s):
    return jnp.where(x > 0, x, s * x)


def _ln(x, g, b, eps=1e-5):
    m = x.mean(-1, keepdims=True)
    v = ((x - m) ** 2).mean(-1, keepdims=True)
    return (x - m) / jnp.sqrt(v + eps) * g + b


def _sc_logit_body(src_hbm, dst_hbm, ae_hbm, as_hbm, ad_hbm, ed_out,
                   as_v, ad_v, srcb, dstb, aeb, edb):
    c = lax.axis_index("c")
    s = lax.axis_index("s")
    w = s * 2 + c
    iot = lax.iota(jnp.int32, 16)
    pltpu.sync_copy(as_hbm, as_v)
    pltpu.sync_copy(ad_hbm, ad_v)

    def chunk(g, _):
        off = w * PER_A + g * CHA
        pltpu.sync_copy(src_hbm.at[pl.ds(off, CHA)], srcb)
        pltpu.sync_copy(dst_hbm.at[pl.ds(off, CHA)], dstb)
        pltpu.sync_copy(ae_hbm.at[pl.ds(off, CHA)], aeb)
        for j in range(CHA // 16):
            sv = srcb[pl.ds(j * 16, 16)]
            dv = dstb[pl.ds(j * 16, 16)]
            a = (plsc.load_gather(as_v, [sv]) + plsc.load_gather(ad_v, [dv])
                 + aeb[pl.ds(j * 16, 16)])
            a = jnp.where(a > 0, a, a * jnp.float32(0.2))
            exi = plsc.bitcast(jnp.exp(a), jnp.int32)
            pos = iot * 4 + j * 64
            plsc.store_scatter(edb, [pos], sv)
            plsc.store_scatter(edb, [pos + 1], dv)
            plsc.store_scatter(edb, [pos + 2], exi)
        pltpu.sync_copy(edb, ed_out.at[pl.ds(off * 4, CHA * 4)])
        return 0
    lax.fori_loop(0, PER_A // CHA, chunk, 0)


_sc_logit = functools.partial(
    pl.kernel,
    out_type=jax.ShapeDtypeStruct((EPAD4,), jnp.int32),
    mesh=plsc.VectorSubcoreMesh(core_axis_name="c", subcore_axis_name="s"),
    compiler_params=pltpu.CompilerParams(needs_layout_passes=False,
                                         use_tc_tiling_on_sc=False),
    scratch_types=[
        pltpu.VMEM((NPAD,), jnp.float32),
        pltpu.VMEM((NPAD,), jnp.float32),
        pltpu.VMEM((CHA,), jnp.int32),
        pltpu.VMEM((CHA,), jnp.int32),
        pltpu.VMEM((CHA,), jnp.float32),
        pltpu.VMEM((CHA * 4,), jnp.int32),
    ],
)(_sc_logit_body)


def _sc_agg_body(ed_hbm, hs_hbm, acc_out,
                 edata, srcb, idxb, exmb, rows, acc_sh,
                 sem_l, sem_g, sem_s):
    c = lax.axis_index("c")
    s = lax.axis_index("s")
    base = c * H2
    iot = lax.iota(jnp.int32, 16)
    zf = jnp.zeros((16,), jnp.float32)
    ebase = s * PER_SUB * 4

    def zrow(i, _):
        for b in range(2):
            for k in range(W80 // 16):
                rows[b, i, pl.ds(k * 16, 16)] = zf
        return 0
    lax.fori_loop(0, CHB, zrow, 0)

    def zfill(i, _):
        t = i * NSUB + s
        pltpu.sync_copy(rows.at[0], acc_sh.at[pl.ds(t * CHB, CHB)])
        return 0
    lax.fori_loop(0, NZCH // NSUB, zfill, 0)
    plsc.subcore_barrier()

    def lin_start(g, b):
        pltpu.async_copy(ed_hbm.at[pl.ds(ebase + g * CHB * 4, CHB * 4)],
                         edata.at[b], sem_l.at[b])

    def lin_wait(g, b):
        pltpu.make_async_copy(ed_hbm.at[pl.ds(ebase + g * CHB * 4, CHB * 4)],
                              edata.at[b], sem_l.at[b]).wait()

    def gath_start(b):
        pltpu.async_copy(hs_hbm.at[srcb.at[b]], rows.at[b], sem_g.at[b])

    def gath_wait(b):
        pltpu.make_async_copy(hs_hbm.at[srcb.at[b]], rows.at[b],
                              sem_g.at[b]).wait()

    def scat_start(b):
        pltpu.async_copy(rows.at[b], acc_sh.at[idxb.at[b]], sem_s.at[b],
                         add=True)

    def scat_wait(b):
        pltpu.make_async_copy(rows.at[b], acc_sh.at[idxb.at[b]],
                              sem_s.at[b]).wait()

    def decode(b):
        ed = edata.at[b]
        for j in range(CHB // 16):
            pos = iot * 4 + j * 64
            sv = plsc.load_gather(ed, [pos])
            dv = plsc.load_gather(ed, [pos + 1])
            exv = plsc.bitcast(plsc.load_gather(ed, [pos + 2]), jnp.float32)
            own = (dv >= base) & (dv < base + H2)
            srcb[b, pl.ds(j * 16, 16)] = sv
            idxb[b, pl.ds(j * 16, 16)] = jnp.where(own, dv - base, 0)
            exmb[b, pl.ds(j * 16, 16)] = jnp.where(own, exv, jnp.float32(0.0))

    def scale(b):
        for j in range(CHB // 16):
            exm = exmb[b, pl.ds(j * 16, 16)]
            for l in range(16):
                sx = exm[l]
                e = j * 16 + l
                for k in range(W80 // 16):
                    rows[b, e, pl.ds(k * 16, 16)] = (
                        rows[b, e, pl.ds(k * 16, 16)] * sx)

    lin_start(0, 0)

    def outer(g2, _):
        for b in range(2):
            g = g2 * 2 + b
            lin_wait(g, b)
            @pl.when(g >= 2)
            def _():
                scat_wait(b)
            decode(b)
            gath_start(b)
            @pl.when(g + 1 < NCHB)
            def _():
                lin_start(g + 1, 1 - b)
            @pl.when(g >= 1)
            def _():
                gath_wait(1 - b)
                scale(1 - b)
                scat_start(1 - b)
        return 0
    lax.fori_loop(0, NCHB // 2, outer, 0)
    gath_wait(1)
    scale(1)
    scat_start(1)
    scat_wait(0)
    scat_wait(1)
    plsc.subcore_barrier()

    def wb(i, _):
        t = i * NSUB + s
        pltpu.sync_copy(acc_sh.at[pl.ds(t * CHB, CHB)], rows.at[0])
        pltpu.sync_copy(rows.at[0], acc_out.at[c, pl.ds(t * CHB, CHB)])
        return 0
    lax.fori_loop(0, NZCH // NSUB, wb, 0)


_sc_agg = functools.partial(
    pl.kernel,
    out_type=jax.ShapeDtypeStruct((2, H2, W80), jnp.float32),
    mesh=plsc.VectorSubcoreMesh(core_axis_name="c", subcore_axis_name="s"),
    compiler_params=pltpu.CompilerParams(needs_layout_passes=False,
                                         use_tc_tiling_on_sc=False),
    scratch_types=[
        pltpu.VMEM((2, CHB * 4), jnp.int32),
        pltpu.VMEM((2, CHB), jnp.int32),
        pltpu.VMEM((2, CHB), jnp.int32),
        pltpu.VMEM((2, CHB), jnp.float32),
        pltpu.VMEM((2, CHB, W80), jnp.float32),
        pltpu.VMEM_SHARED((H2, W80), jnp.float32),
        pltpu.SemaphoreType.DMA((2,)),
        pltpu.SemaphoreType.DMA((2,)),
        pltpu.SemaphoreType.DMA((2,)),
    ],
)(_sc_agg_body)


def _padn(x):
    return jnp.zeros((NPAD,) + x.shape[1:], x.dtype).at[:x.shape[0]].set(x)


def _pade(x, fill):
    return jnp.concatenate(
        [x, jnp.full((EPAD - E,) + x.shape[1:], fill, x.dtype)])


_BLK = 512
_GRID = NPAD // _BLK


def _prep_body(xs_ref, xd_ref, w_ref, av_ref, u_ref, hs80_ref, asad_ref):
    hs = jnp.dot(xs_ref[...], w_ref[...], preferred_element_type=jnp.float32)
    col16 = lax.broadcasted_iota(jnp.int32, (_BLK, 16), 1)
    tail = jnp.where(col16 == 0, jnp.float32(1.0), jnp.float32(0.0))
    hs80_ref[...] = lax.concatenate([hs, tail], 1)
    a_s = jnp.sum(hs * av_ref[...], axis=1, keepdims=True)
    a_d = jnp.sum(xd_ref[...] * u_ref[...], axis=1, keepdims=True)
    col8 = lax.broadcasted_iota(jnp.int32, (_BLK, 8), 1)
    asad_ref[...] = jnp.where(col8 == 0, a_s,
                              jnp.where(col8 == 1, a_d, jnp.float32(0.0)))


def _prep_tc(xs, xd, W_src, avec, u):
    ds, dd = xs.shape[1], xd.shape[1]
    hs80, asad = pl.pallas_call(
        _prep_body,
        grid=(_GRID,),
        in_specs=[
            pl.BlockSpec((_BLK, ds), lambda i: (i, 0)),
            pl.BlockSpec((_BLK, dd), lambda i: (i, 0)),
            pl.BlockSpec((ds, HID), lambda i: (0, 0)),
            pl.BlockSpec((1, HID), lambda i: (0, 0)),
            pl.BlockSpec((1, dd), lambda i: (0, 0)),
        ],
        out_specs=[pl.BlockSpec((_BLK, W80), lambda i: (i, 0)),
                   pl.BlockSpec((_BLK, 8), lambda i: (i, 0))],
        out_shape=[jax.ShapeDtypeStruct((NPAD, W80), jnp.float32),
                   jax.ShapeDtypeStruct((NPAD, 8), jnp.float32)],
    )(xs, xd, W_src, avec, u)
    return hs80, asad[:, 0], asad[:, 1]


def _post_body(acc_ref, xd_ref, wr_ref, b_ref, g_ref, bn_ref, out_ref, *,
               mode):
    num = acc_ref[:, :HID]
    den = acc_ref[:, HID:HID + 1]
    o = (num / (den + 1e-16)
         + jnp.dot(xd_ref[...], wr_ref[...],
                   preferred_element_type=jnp.float32) + b_ref[...])
    if mode == 'act_ln':
        o = _ln(_leaky(o, 0.01), g_ref[...], bn_ref[...])
    elif mode == 'ln_act':
        o = _leaky(_ln(o, g_ref[...], bn_ref[...]), 0.01)
    out_ref[...] = o


def _post_tc(acc, xd, W_res, b, g, bn, mode):
    dd = xd.shape[1]
    return pl.pallas_call(
        functools.partial(_post_body, mode=mode),
        grid=(_GRID,),
        in_specs=[
            pl.BlockSpec((_BLK, W80), lambda i: (i, 0)),
            pl.BlockSpec((_BLK, dd), lambda i: (i, 0)),
            pl.BlockSpec((dd, HID), lambda i: (0, 0)),
            pl.BlockSpec((1, HID), lambda i: (0, 0)),
            pl.BlockSpec((1, HID), lambda i: (0, 0)),
            pl.BlockSpec((1, HID), lambda i: (0, 0)),
        ],
        out_specs=pl.BlockSpec((_BLK, HID), lambda i: (i, 0)),
        out_shape=jax.ShapeDtypeStruct((NPAD, HID), jnp.float32),
    )(acc, xd, W_res, b[None], g[None], bn[None])


_GE = EPAD // 128
_BE = 392


def _ae_body(d0, d1, d2, t0, t1, t2, u0, coef, *outs):
    def w(r, j):
        return coef[r:r + 1, j:j + 1]
    cols = [(d0, d1, d2), (d0, d1, d2), (t0, t1, t2),
            (u0,), (u0,), (u0,), (u0,)]
    for r, (o_ref, cs) in enumerate(zip(outs, cols)):
        acc = cs[0][...] * w(r, 0)
        for j in range(1, len(cs)):
            acc = acc + cs[j][...] * w(r, j)
        o_ref[...] = acc


def _ae_tc(cdt, ctd, ctt, coef):
    blk = pl.BlockSpec((_BE, 128), lambda i: (i, 0))
    outs = pl.pallas_call(
        _ae_body,
        grid=(_GE // _BE,),
        in_specs=[blk] * 6 + [blk,
                              pl.BlockSpec((8, 128), lambda i: (0, 0))],
        out_specs=[blk] * 7,
        out_shape=[jax.ShapeDtypeStruct((_GE, 128), jnp.float32)] * 7,
    )(*cdt, *ctd, ctt, coef)
    return [o.reshape(EPAD) for o in outs]


def _head1_body(dep_ref, dpt_ref, t2_ref, wxa, wxb, bx, gx, bnx,
                wy, by, gy, bny, wc, bc, gc, bnc, z_ref, psum_ref):
    i = pl.program_id(0)
    dot = lambda a, b: jnp.dot(a, b, preferred_element_type=jnp.float32)
    zx = _leaky(_ln(dot(dep_ref[...], wxa[...]) + dot(dpt_ref[...], wxb[...])
                    + bx[...], gx[...], bnx[...]), 0.01)
    zy = _leaky(_ln(dot(t2_ref[...], wy[...]) + by[...], gy[...], bny[...]),
                0.01)
    z2 = _leaky(_ln(dot(zx + zy, wc[...]) + bc[...], gc[...], bnc[...]), 0.01)
    z_ref[...] = z2
    rowid = i * _BLK + lax.broadcasted_iota(jnp.int32, (_BLK, 1), 0)
    masked = jnp.where(rowid < NT, z2, jnp.float32(0.0))

    @pl.when(i == 0)
    def _():
        psum_ref[...] = jnp.zeros_like(psum_ref)
    psum_ref[...] += jnp.sum(masked, axis=0, keepdims=True)


def _head1_tc(dep, dpt, t2, P):
    row = pl.BlockSpec((_BLK, HID), lambda i: (i, 0))
    w64 = pl.BlockSpec((HID, HID), lambda i: (0, 0))
    v64 = pl.BlockSpec((1, HID), lambda i: (0, 0))
    z, psum = pl.pallas_call(
        _head1_body,
        grid=(_GRID,),
        in_specs=[row, row, row] + [w64, w64, v64, v64, v64,
                                    w64, v64, v64, v64,
                                    w64, v64, v64, v64],
        out_specs=[row, pl.BlockSpec((1, HID), lambda i: (0, 0))],
        out_shape=[jax.ShapeDtypeStruct((NPAD, HID), jnp.float32),
                   jax.ShapeDtypeStruct((1, HID), jnp.float32)],
    )(dep, dpt, t2,
      P['fc_x']['W'][:HID], P['fc_x']['W'][HID:], P['fc_x']['b'][None],
      P['ln_x']['g'][None], P['ln_x']['b'][None],
      P['fc_y']['W'], P['fc_y']['b'][None],
      P['ln_y']['g'][None], P['ln_y']['b'][None],
      P['fc_c']['W'], P['fc_c']['b'][None],
      P['ln_c']['g'][None], P['ln_c']['b'][None])
    return z, psum


def _gat_sc(hs80, a_s, a_d, edges, ae, x_dst, p, ln, mode):
    ed = _sc_logit(edges[0], edges[1], ae, a_s, a_d)
    acc = _sc_agg(ed, hs80)
    acc = acc.reshape(2 * H2, W80)
    return _post_tc(acc, x_dst, p['W_res'], p['b'], ln[0], ln[1], mode)


def _head2_body(z2_ref, zsum_ref, w1a_ref, w1b_ref, b1_ref, g1_ref, bn1_ref,
                w2_ref, b2_ref, out_ref):
    r = jnp.dot(zsum_ref[...] * jnp.float32(1.0 / NT), w1b_ref[...],
                preferred_element_type=jnp.float32)
    h = (jnp.dot(z2_ref[...], w1a_ref[...],
                 preferred_element_type=jnp.float32) + r + b1_ref[...])
    h = _ln(h, g1_ref[...], bn1_ref[...])
    h = _leaky(h, 0.01)
    out_ref[...] = jnp.dot(h, w2_ref[...],
                           preferred_element_type=jnp.float32) + b2_ref[...]


def _head2(z2, zsum, w1a, w1b, b1, g1, bn1, w2p, b2p):
    return pl.pallas_call(
        _head2_body,
        grid=(_GRID,),
        in_specs=[
            pl.BlockSpec((_BLK, HID), lambda i: (i, 0)),
            pl.BlockSpec((1, HID), lambda i: (0, 0)),
            pl.BlockSpec((HID, HID), lambda i: (0, 0)),
            pl.BlockSpec((HID, HID), lambda i: (0, 0)),
            pl.BlockSpec((1, HID), lambda i: (0, 0)),
            pl.BlockSpec((1, HID), lambda i: (0, 0)),
            pl.BlockSpec((1, HID), lambda i: (0, 0)),
            pl.BlockSpec((HID, 128), lambda i: (0, 0)),
            pl.BlockSpec((1, 128), lambda i: (0, 0)),
        ],
        out_specs=pl.BlockSpec((_BLK, 128), lambda i: (i, 0)),
        out_shape=jax.ShapeDtypeStruct((NPAD, 128), jnp.float32),
    )(z2, zsum, w1a, w1b, b1, g1, bn1, w2p, b2p)


def kernel(x_tasks, x_data, ea_dt, ea_td, ea_tt, params, ei_dt, ei_td, ei_tt):
    P = params

    e_dt = (_pade(ei_dt[0], 0), _pade(ei_dt[1], NT))
    e_td = (_pade(ei_td[0], 0), _pade(ei_td[1], NT))
    e_tt = (_pade(ei_tt[0], 0), _pade(ei_tt[1], NT))
    e_ttf = (e_tt[1], _pade(ei_tt[0], NT))

    def cols3(ea):
        return [_pade(ea[:, j], 0.0).reshape(_GE, 128) for j in range(3)]

    def evec(p):
        return p['W_edge'] @ p['att_edge'][0]

    coef = jnp.zeros((8, 128), jnp.float32)
    coef = coef.at[0, :3].set(evec(P['dt1']))
    coef = coef.at[1, :3].set(evec(P['dt2']))
    coef = coef.at[2, :3].set(evec(P['td1']))
    coef = coef.at[3, :1].set(evec(P['dep1']))
    coef = coef.at[4, :1].set(evec(P['dpt1']))
    coef = coef.at[5, :1].set(evec(P['dep2']))
    coef = coef.at[6, :1].set(evec(P['dpt2']))
    ctt = _pade(ea_tt[:, 0], 0.0).reshape(_GE, 128)
    (ae_dt1, ae_dt2, ae_td1, ae_dep1,
     ae_dpt1, ae_dep2, ae_dpt2) = _ae_tc(cols3(ea_dt), cols3(ea_td),
                                         ctt, coef)

    def avec_of(p):
        return p['att_src'][0][None]

    def u_of(p):
        return (p['W_dst'] @ p['att_dst'][0])[None]

    def gat(xs, xd, p, edges, ae, ln, mode):
        hs80, a_s, a_d = _prep_tc(xs, xd, p['W_src'], avec_of(p), u_of(p))
        return _gat_sc(hs80, a_s, a_d, edges, ae, xd, p, ln, mode)

    xt = _padn(x_tasks)
    xd = _padn(x_data)
    t1 = gat(xd, xt, P['dt1'], e_dt, ae_dt1,
             (P['ln_t']['g'], P['ln_t']['b']), 'act_ln')
    d1 = gat(xt, xd, P['td1'], e_td, ae_td1,
             (P['ln_d']['g'], P['ln_d']['b']), 'act_ln')
    t2 = gat(d1, t1, P['dt2'], e_dt, ae_dt2,
             (P['ln_t']['g'], P['ln_t']['b']), 'none')
    dep = gat(t2, t2, P['dep1'], e_tt, ae_dep1,
              (P['ln_dep']['g'], P['ln_dep']['b']), 'ln_act')
    dpt = gat(t2, t2, P['dpt1'], e_ttf, ae_dpt1,
              (P['ln_dpt']['g'], P['ln_dpt']['b']), 'ln_act')
    dep = gat(dep, dep, P['dep2'], e_tt, ae_dep2,
              (P['ln_dep']['g'], P['ln_dep']['b']), 'none')
    dpt = gat(dpt, dpt, P['dpt2'], e_ttf, ae_dpt2,
              (P['ln_dpt']['g'], P['ln_dpt']['b']), 'none')

    z, zsum = _head1_tc(dep, dpt, t2, P)
    w2p = jnp.zeros((HID, 128), jnp.float32).at[:, :NDEV].set(P['fc2']['W'])
    b2p = jnp.zeros((1, 128), jnp.float32).at[0, :NDEV].set(P['fc2']['b'])
    out = _head2(z, zsum, P['fc1']['W'][:HID], P['fc1']['W'][HID:],
                 P['fc1']['b'][None], P['ln1']['g'][None],
                 P['ln1']['b'][None], w2p, b2p)
    return out[:NT, :NDEV]


# speedup vs baseline: 1.0317x; 1.0317x over previous
import functools

import jax
import jax.numpy as jnp
from jax import lax
from jax.experimental import pallas as pl
from jax.experimental.pallas import tpu as pltpu
from jax.experimental.pallas import tpu_sc as plsc

HID = 64
W80 = 80
NDEV = 5
NT = 50000
NPAD = 50176
H2 = NPAD // 2
E = 800000
NSUB = 16
PER_SUB = 50176
EPAD = PER_SUB * NSUB
EPAD4 = EPAD * 4
PER_A = EPAD // 32
CHA = 256
CHB = 32
NCHB = PER_SUB // CHB
NZCH = H2 // CHB


def _leaky(x, s):
    return jnp.where(x > 0, x, s * x)


def _ln(x, g, b, eps=1e-5):
    m = x.mean(-1, keepdims=True)
    v = ((x - m) ** 2).mean(-1, keepdims=True)
    return (x - m) / jnp.sqrt(v + eps) * g + b


def _sc_logit_body(src_hbm, dst_hbm, ae_hbm, as_hbm, ad_hbm, ed_out,
                   as_v, ad_v, srcb, dstb, aeb, edb,
                   sem_i, sem_o):
    c = lax.axis_index("c")
    s = lax.axis_index("s")
    w = s * 2 + c
    iot = lax.iota(jnp.int32, 16)
    nch = PER_A // CHA
    pltpu.sync_copy(as_hbm, as_v)
    pltpu.sync_copy(ad_hbm, ad_v)

    def in_start(g, b):
        off = w * PER_A + g * CHA
        pltpu.async_copy(src_hbm.at[pl.ds(off, CHA)], srcb.at[b], sem_i.at[b])
        pltpu.async_copy(dst_hbm.at[pl.ds(off, CHA)], dstb.at[b], sem_i.at[b])
        pltpu.async_copy(ae_hbm.at[pl.ds(off, CHA)], aeb.at[b], sem_i.at[b])

    def in_wait(g, b):
        off = w * PER_A + g * CHA
        pltpu.make_async_copy(src_hbm.at[pl.ds(off, CHA)], srcb.at[b],
                              sem_i.at[b]).wait()
        pltpu.make_async_copy(dst_hbm.at[pl.ds(off, CHA)], dstb.at[b],
                              sem_i.at[b]).wait()
        pltpu.make_async_copy(ae_hbm.at[pl.ds(off, CHA)], aeb.at[b],
                              sem_i.at[b]).wait()

    def out_start(g, b):
        off = w * PER_A + g * CHA
        pltpu.async_copy(edb.at[b], ed_out.at[pl.ds(off * 4, CHA * 4)],
                         sem_o.at[b])

    def out_wait(g, b):
        off = w * PER_A + g * CHA
        pltpu.make_async_copy(edb.at[b], ed_out.at[pl.ds(off * 4, CHA * 4)],
                              sem_o.at[b]).wait()

    in_start(0, 0)

    def outer(g2, _):
        for b in range(2):
            g = g2 * 2 + b
            in_wait(g, b)
            @pl.when(g + 1 < nch)
            def _():
                in_start(g + 1, 1 - b)
            @pl.when(g >= 2)
            def _():
                out_wait(g - 2, b)
            for j in range(CHA // 16):
                sv = srcb[b, pl.ds(j * 16, 16)]
                dv = dstb[b, pl.ds(j * 16, 16)]
                a = (plsc.load_gather(as_v, [sv])
                     + plsc.load_gather(ad_v, [dv])
                     + aeb[b, pl.ds(j * 16, 16)])
                a = jnp.where(a > 0, a, a * jnp.float32(0.2))
                exi = plsc.bitcast(jnp.exp(a), jnp.int32)
                pos = iot * 4 + j * 64
                plsc.store_scatter(edb.at[b], [pos], sv)
                plsc.store_scatter(edb.at[b], [pos + 1], dv)
                plsc.store_scatter(edb.at[b], [pos + 2], exi)
            out_start(g, b)
        return 0
    lax.fori_loop(0, nch // 2, outer, 0)
    out_wait(nch - 2, 0)
    out_wait(nch - 1, 1)


_sc_logit = functools.partial(
    pl.kernel,
    out_type=jax.ShapeDtypeStruct((EPAD4,), jnp.int32),
    mesh=plsc.VectorSubcoreMesh(core_axis_name="c", subcore_axis_name="s"),
    compiler_params=pltpu.CompilerParams(needs_layout_passes=False,
                                         use_tc_tiling_on_sc=False),
    scratch_types=[
        pltpu.VMEM((NPAD,), jnp.float32),
        pltpu.VMEM((NPAD,), jnp.float32),
        pltpu.VMEM((2, CHA), jnp.int32),
        pltpu.VMEM((2, CHA), jnp.int32),
        pltpu.VMEM((2, CHA), jnp.float32),
        pltpu.VMEM((2, CHA * 4), jnp.int32),
        pltpu.SemaphoreType.DMA((2,)),
        pltpu.SemaphoreType.DMA((2,)),
    ],
)(_sc_logit_body)


def _sc_agg_body(ed_hbm, hs_hbm, acc_out,
                 edata, srcb, idxb, exmb, rows, acc_sh,
                 sem_l, sem_g, sem_s):
    c = lax.axis_index("c")
    s = lax.axis_index("s")
    base = c * H2
    iot = lax.iota(jnp.int32, 16)
    zf = jnp.zeros((16,), jnp.float32)
    ebase = s * PER_SUB * 4

    def zrow(i, _):
        for b in range(2):
            for k in range(W80 // 16):
                rows[b, i, pl.ds(k * 16, 16)] = zf
        return 0
    lax.fori_loop(0, CHB, zrow, 0)

    def zfill(i, _):
        t = i * NSUB + s
        pltpu.sync_copy(rows.at[0], acc_sh.at[pl.ds(t * CHB, CHB)])
        return 0
    lax.fori_loop(0, NZCH // NSUB, zfill, 0)
    plsc.subcore_barrier()

    def lin_start(g, b):
        pltpu.async_copy(ed_hbm.at[pl.ds(ebase + g * CHB * 4, CHB * 4)],
                         edata.at[b], sem_l.at[b])

    def lin_wait(g, b):
        pltpu.make_async_copy(ed_hbm.at[pl.ds(ebase + g * CHB * 4, CHB * 4)],
                              edata.at[b], sem_l.at[b]).wait()

    def gath_start(b):
        pltpu.async_copy(hs_hbm.at[srcb.at[b]], rows.at[b], sem_g.at[b])

    def gath_wait(b):
        pltpu.make_async_copy(hs_hbm.at[srcb.at[b]], rows.at[b],
                              sem_g.at[b]).wait()

    def scat_start(b):
        pltpu.async_copy(rows.at[b], acc_sh.at[idxb.at[b]], sem_s.at[b],
                         add=True)

    def scat_wait(b):
        pltpu.make_async_copy(rows.at[b], acc_sh.at[idxb.at[b]],
                              sem_s.at[b]).wait()

    def decode(b):
        ed = edata.at[b]
        for j in range(CHB // 16):
            pos = iot * 4 + j * 64
            sv = plsc.load_gather(ed, [pos])
            dv = plsc.load_gather(ed, [pos + 1])
            exv = plsc.bitcast(plsc.load_gather(ed, [pos + 2]), jnp.float32)
            own = (dv >= base) & (dv < base + H2)
            srcb[b, pl.ds(j * 16, 16)] = sv
            idxb[b, pl.ds(j * 16, 16)] = jnp.where(own, dv - base, 0)
            exmb[b, pl.ds(j * 16, 16)] = jnp.where(own, exv, jnp.float32(0.0))

    def scale(b):
        for j in range(CHB // 16):
            exm = exmb[b, pl.ds(j * 16, 16)]
            for l in range(16):
                sx = exm[l]
                e = j * 16 + l
                for k in range(W80 // 16):
                    rows[b, e, pl.ds(k * 16, 16)] = (
                        rows[b, e, pl.ds(k * 16, 16)] * sx)

    lin_start(0, 0)

    def outer(g2, _):
        for b in range(2):
            g = g2 * 2 + b
            lin_wait(g, b)
            @pl.when(g >= 2)
            def _():
                scat_wait(b)
            decode(b)
            gath_start(b)
            @pl.when(g + 1 < NCHB)
            def _():
                lin_start(g + 1, 1 - b)
            @pl.when(g >= 1)
            def _():
                gath_wait(1 - b)
                scale(1 - b)
                scat_start(1 - b)
        return 0
    lax.fori_loop(0, NCHB // 2, outer, 0)
    gath_wait(1)
    scale(1)
    scat_start(1)
    scat_wait(0)
    scat_wait(1)
    plsc.subcore_barrier()

    def wb(i, _):
        t = i * NSUB + s
        pltpu.sync_copy(acc_sh.at[pl.ds(t * CHB, CHB)], rows.at[0])
        pltpu.sync_copy(rows.at[0], acc_out.at[c, pl.ds(t * CHB, CHB)])
        return 0
    lax.fori_loop(0, NZCH // NSUB, wb, 0)


_sc_agg = functools.partial(
    pl.kernel,
    out_type=jax.ShapeDtypeStruct((2, H2, W80), jnp.float32),
    mesh=plsc.VectorSubcoreMesh(core_axis_name="c", subcore_axis_name="s"),
    compiler_params=pltpu.CompilerParams(needs_layout_passes=False,
                                         use_tc_tiling_on_sc=False),
    scratch_types=[
        pltpu.VMEM((2, CHB * 4), jnp.int32),
        pltpu.VMEM((2, CHB), jnp.int32),
        pltpu.VMEM((2, CHB), jnp.int32),
        pltpu.VMEM((2, CHB), jnp.float32),
        pltpu.VMEM((2, CHB, W80), jnp.float32),
        pltpu.VMEM_SHARED((H2, W80), jnp.float32),
        pltpu.SemaphoreType.DMA((2,)),
        pltpu.SemaphoreType.DMA((2,)),
        pltpu.SemaphoreType.DMA((2,)),
    ],
)(_sc_agg_body)


def _padn(x):
    return jnp.zeros((NPAD,) + x.shape[1:], x.dtype).at[:x.shape[0]].set(x)


def _pade(x, fill):
    return jnp.concatenate(
        [x, jnp.full((EPAD - E,) + x.shape[1:], fill, x.dtype)])


_BLK = 512
_GRID = NPAD // _BLK


def _prep_body(xs_ref, xd_ref, w_ref, av_ref, u_ref, hs80_ref, asad_ref):
    hs = jnp.dot(xs_ref[...], w_ref[...], preferred_element_type=jnp.float32)
    col16 = lax.broadcasted_iota(jnp.int32, (_BLK, 16), 1)
    tail = jnp.where(col16 == 0, jnp.float32(1.0), jnp.float32(0.0))
    hs80_ref[...] = lax.concatenate([hs, tail], 1)
    a_s = jnp.sum(hs * av_ref[...], axis=1, keepdims=True)
    a_d = jnp.sum(xd_ref[...] * u_ref[...], axis=1, keepdims=True)
    col8 = lax.broadcasted_iota(jnp.int32, (_BLK, 8), 1)
    asad_ref[...] = jnp.where(col8 == 0, a_s,
                              jnp.where(col8 == 1, a_d, jnp.float32(0.0)))


def _prep_tc(xs, xd, W_src, avec, u):
    ds, dd = xs.shape[1], xd.shape[1]
    hs80, asad = pl.pallas_call(
        _prep_body,
        grid=(_GRID,),
        in_specs=[
            pl.BlockSpec((_BLK, ds), lambda i: (i, 0)),
            pl.BlockSpec((_BLK, dd), lambda i: (i, 0)),
            pl.BlockSpec((ds, HID), lambda i: (0, 0)),
            pl.BlockSpec((1, HID), lambda i: (0, 0)),
            pl.BlockSpec((1, dd), lambda i: (0, 0)),
        ],
        out_specs=[pl.BlockSpec((_BLK, W80), lambda i: (i, 0)),
                   pl.BlockSpec((_BLK, 8), lambda i: (i, 0))],
        out_shape=[jax.ShapeDtypeStruct((NPAD, W80), jnp.float32),
                   jax.ShapeDtypeStruct((NPAD, 8), jnp.float32)],
    )(xs, xd, W_src, avec, u)
    return hs80, asad[:, 0], asad[:, 1]


def _post_body(acc_ref, xd_ref, wr_ref, b_ref, g_ref, bn_ref, out_ref, *,
               mode):
    num = acc_ref[:, :HID]
    den = acc_ref[:, HID:HID + 1]
    o = (num / (den + 1e-16)
         + jnp.dot(xd_ref[...], wr_ref[...],
                   preferred_element_type=jnp.float32) + b_ref[...])
    if mode == 'act_ln':
        o = _ln(_leaky(o, 0.01), g_ref[...], bn_ref[...])
    elif mode == 'ln_act':
        o = _leaky(_ln(o, g_ref[...], bn_ref[...]), 0.01)
    out_ref[...] = o


def _post_tc(acc, xd, W_res, b, g, bn, mode):
    dd = xd.shape[1]
    return pl.pallas_call(
        functools.partial(_post_body, mode=mode),
        grid=(_GRID,),
        in_specs=[
            pl.BlockSpec((_BLK, W80), lambda i: (i, 0)),
            pl.BlockSpec((_BLK, dd), lambda i: (i, 0)),
            pl.BlockSpec((dd, HID), lambda i: (0, 0)),
            pl.BlockSpec((1, HID), lambda i: (0, 0)),
            pl.BlockSpec((1, HID), lambda i: (0, 0)),
            pl.BlockSpec((1, HID), lambda i: (0, 0)),
        ],
        out_specs=pl.BlockSpec((_BLK, HID), lambda i: (i, 0)),
        out_shape=jax.ShapeDtypeStruct((NPAD, HID), jnp.float32),
    )(acc, xd, W_res, b[None], g[None], bn[None])


_GE = EPAD // 128
_BE = 392


def _ae_body(d0, d1, d2, t0, t1, t2, u0, coef, *outs):
    def w(r, j):
        return coef[r:r + 1, j:j + 1]
    cols = [(d0, d1, d2), (d0, d1, d2), (t0, t1, t2),
            (u0,), (u0,), (u0,), (u0,)]
    for r, (o_ref, cs) in enumerate(zip(outs, cols)):
        acc = cs[0][...] * w(r, 0)
        for j in range(1, len(cs)):
            acc = acc + cs[j][...] * w(r, j)
        o_ref[...] = acc


def _ae_tc(cdt, ctd, ctt, coef):
    blk = pl.BlockSpec((_BE, 128), lambda i: (i, 0))
    outs = pl.pallas_call(
        _ae_body,
        grid=(_GE // _BE,),
        in_specs=[blk] * 6 + [blk,
                              pl.BlockSpec((8, 128), lambda i: (0, 0))],
        out_specs=[blk] * 7,
        out_shape=[jax.ShapeDtypeStruct((_GE, 128), jnp.float32)] * 7,
    )(*cdt, *ctd, ctt, coef)
    return [o.reshape(EPAD) for o in outs]


def _head1_body(dep_ref, dpt_ref, t2_ref, wxa, wxb, bx, gx, bnx,
                wy, by, gy, bny, wc, bc, gc, bnc, z_ref, psum_ref):
    i = pl.program_id(0)
    dot = lambda a, b: jnp.dot(a, b, preferred_element_type=jnp.float32)
    zx = _leaky(_ln(dot(dep_ref[...], wxa[...]) + dot(dpt_ref[...], wxb[...])
                    + bx[...], gx[...], bnx[...]), 0.01)
    zy = _leaky(_ln(dot(t2_ref[...], wy[...]) + by[...], gy[...], bny[...]),
                0.01)
    z2 = _leaky(_ln(dot(zx + zy, wc[...]) + bc[...], gc[...], bnc[...]), 0.01)
    z_ref[...] = z2
    rowid = i * _BLK + lax.broadcasted_iota(jnp.int32, (_BLK, 1), 0)
    masked = jnp.where(rowid < NT, z2, jnp.float32(0.0))

    @pl.when(i == 0)
    def _():
        psum_ref[...] = jnp.zeros_like(psum_ref)
    psum_ref[...] += jnp.sum(masked, axis=0, keepdims=True)


def _head1_tc(dep, dpt, t2, P):
    row = pl.BlockSpec((_BLK, HID), lambda i: (i, 0))
    w64 = pl.BlockSpec((HID, HID), lambda i: (0, 0))
    v64 = pl.BlockSpec((1, HID), lambda i: (0, 0))
    z, psum = pl.pallas_call(
        _head1_body,
        grid=(_GRID,),
        in_specs=[row, row, row] + [w64, w64, v64, v64, v64,
                                    w64, v64, v64, v64,
                                    w64, v64, v64, v64],
        out_specs=[row, pl.BlockSpec((1, HID), lambda i: (0, 0))],
        out_shape=[jax.ShapeDtypeStruct((NPAD, HID), jnp.float32),
                   jax.ShapeDtypeStruct((1, HID), jnp.float32)],
    )(dep, dpt, t2,
      P['fc_x']['W'][:HID], P['fc_x']['W'][HID:], P['fc_x']['b'][None],
      P['ln_x']['g'][None], P['ln_x']['b'][None],
      P['fc_y']['W'], P['fc_y']['b'][None],
      P['ln_y']['g'][None], P['ln_y']['b'][None],
      P['fc_c']['W'], P['fc_c']['b'][None],
      P['ln_c']['g'][None], P['ln_c']['b'][None])
    return z, psum


def _gat_sc(hs80, a_s, a_d, edges, ae, x_dst, p, ln, mode):
    ed = _sc_logit(edges[0], edges[1], ae, a_s, a_d)
    acc = _sc_agg(ed, hs80)
    acc = acc.reshape(2 * H2, W80)
    return _post_tc(acc, x_dst, p['W_res'], p['b'], ln[0], ln[1], mode)


def _head2_body(z2_ref, zsum_ref, w1a_ref, w1b_ref, b1_ref, g1_ref, bn1_ref,
                w2_ref, b2_ref, out_ref):
    r = jnp.dot(zsum_ref[...] * jnp.float32(1.0 / NT), w1b_ref[...],
                preferred_element_type=jnp.float32)
    h = (jnp.dot(z2_ref[...], w1a_ref[...],
                 preferred_element_type=jnp.float32) + r + b1_ref[...])
    h = _ln(h, g1_ref[...], bn1_ref[...])
    h = _leaky(h, 0.01)
    out_ref[...] = jnp.dot(h, w2_ref[...],
                           preferred_element_type=jnp.float32) + b2_ref[...]


def _head2(z2, zsum, w1a, w1b, b1, g1, bn1, w2p, b2p):
    return pl.pallas_call(
        _head2_body,
        grid=(_GRID,),
        in_specs=[
            pl.BlockSpec((_BLK, HID), lambda i: (i, 0)),
            pl.BlockSpec((1, HID), lambda i: (0, 0)),
            pl.BlockSpec((HID, HID), lambda i: (0, 0)),
            pl.BlockSpec((HID, HID), lambda i: (0, 0)),
            pl.BlockSpec((1, HID), lambda i: (0, 0)),
            pl.BlockSpec((1, HID), lambda i: (0, 0)),
            pl.BlockSpec((1, HID), lambda i: (0, 0)),
            pl.BlockSpec((HID, 128), lambda i: (0, 0)),
            pl.BlockSpec((1, 128), lambda i: (0, 0)),
        ],
        out_specs=pl.BlockSpec((_BLK, 128), lambda i: (i, 0)),
        out_shape=jax.ShapeDtypeStruct((NPAD, 128), jnp.float32),
    )(z2, zsum, w1a, w1b, b1, g1, bn1, w2p, b2p)


def kernel(x_tasks, x_data, ea_dt, ea_td, ea_tt, params, ei_dt, ei_td, ei_tt):
    P = params

    e_dt = (_pade(ei_dt[0], 0), _pade(ei_dt[1], NT))
    e_td = (_pade(ei_td[0], 0), _pade(ei_td[1], NT))
    e_tt = (_pade(ei_tt[0], 0), _pade(ei_tt[1], NT))
    e_ttf = (e_tt[1], _pade(ei_tt[0], NT))

    def cols3(ea):
        return [_pade(ea[:, j], 0.0).reshape(_GE, 128) for j in range(3)]

    def evec(p):
        return p['W_edge'] @ p['att_edge'][0]

    coef = jnp.zeros((8, 128), jnp.float32)
    coef = coef.at[0, :3].set(evec(P['dt1']))
    coef = coef.at[1, :3].set(evec(P['dt2']))
    coef = coef.at[2, :3].set(evec(P['td1']))
    coef = coef.at[3, :1].set(evec(P['dep1']))
    coef = coef.at[4, :1].set(evec(P['dpt1']))
    coef = coef.at[5, :1].set(evec(P['dep2']))
    coef = coef.at[6, :1].set(evec(P['dpt2']))
    ctt = _pade(ea_tt[:, 0], 0.0).reshape(_GE, 128)
    (ae_dt1, ae_dt2, ae_td1, ae_dep1,
     ae_dpt1, ae_dep2, ae_dpt2) = _ae_tc(cols3(ea_dt), cols3(ea_td),
                                         ctt, coef)

    def avec_of(p):
        return p['att_src'][0][None]

    def u_of(p):
        return (p['W_dst'] @ p['att_dst'][0])[None]

    def gat(xs, xd, p, edges, ae, ln, mode):
        hs80, a_s, a_d = _prep_tc(xs, xd, p['W_src'], avec_of(p), u_of(p))
        return _gat_sc(hs80, a_s, a_d, edges, ae, xd, p, ln, mode)

    xt = _padn(x_tasks)
    xd = _padn(x_data)
    t1 = gat(xd, xt, P['dt1'], e_dt, ae_dt1,
             (P['ln_t']['g'], P['ln_t']['b']), 'act_ln')
    d1 = gat(xt, xd, P['td1'], e_td, ae_td1,
             (P['ln_d']['g'], P['ln_d']['b']), 'act_ln')
    t2 = gat(d1, t1, P['dt2'], e_dt, ae_dt2,
             (P['ln_t']['g'], P['ln_t']['b']), 'none')
    dep = gat(t2, t2, P['dep1'], e_tt, ae_dep1,
              (P['ln_dep']['g'], P['ln_dep']['b']), 'ln_act')
    dpt = gat(t2, t2, P['dpt1'], e_ttf, ae_dpt1,
              (P['ln_dpt']['g'], P['ln_dpt']['b']), 'ln_act')
    dep = gat(dep, dep, P['dep2'], e_tt, ae_dep2,
              (P['ln_dep']['g'], P['ln_dep']['b']), 'none')
    dpt = gat(dpt, dpt, P['dpt2'], e_ttf, ae_dpt2,
              (P['ln_dpt']['g'], P['ln_dpt']['b']), 'none')

    z, zsum = _head1_tc(dep, dpt, t2, P)
    w2p = jnp.zeros((HID, 128), jnp.float32).at[:, :NDEV].set(P['fc2']['W'])
    b2p = jnp.zeros((1, 128), jnp.float32).at[0, :NDEV].set(P['fc2']['b'])
    out = _head2(z, zsum, P['fc1']['W'][:HID], P['fc1']['W'][HID:],
                 P['fc1']['b'][None], P['ln1']['g'][None],
                 P['ln1']['b'][None], w2p, b2p)
    return out[:NT, :NDEV]
